# baseline (device time: 18246 ns/iter reference)
import jax
import jax.numpy as jnp
from jax import lax
from jax.experimental import pallas as pl
from jax.experimental.pallas import tpu as pltpu

N_DEV = 8


def kernel(A, B):
    m_per, k = A.shape
    n = B.shape[1]

    def body(a_ref, b_ref, out_ref, comm_ref, scale_ref, b16_ref,
             ss, rs, ss2, rs2):
        my = lax.axis_index("i")
        z = my // 4
        r = my % 4
        y = r // 2
        x = (r // 2 + r) % 2

        def lid(xx, yy, zz):
            return zz * 4 + yy * 2 + (xx + yy) % 2

        nx = lid(1 - x, y, z)
        ny = lid(x, 1 - y, z)
        nz = lid(x, y, 1 - z)
        anti = lid(1 - x, 1 - y, 1 - z)
        dxy = lid(1 - x, 1 - y, z)
        dyz = lid(x, 1 - y, 1 - z)
        dzx = lid(1 - x, y, 1 - z)

        barrier_sem = pltpu.get_barrier_semaphore()
        for nbr in (nx, ny, nz, anti):
            pl.semaphore_signal(
                barrier_sem,
                inc=1,
                device_id=(nbr,),
                device_id_type=pl.DeviceIdType.MESH,
            )

        a = a_ref[:, :]
        rowmax = jnp.max(jnp.abs(a), axis=1, keepdims=True)
        rowmax = jnp.maximum(rowmax, 1e-30)
        comm_ref[0, :, :] = jnp.rint(a * (127.0 / rowmax)).astype(jnp.int8)
        scale_ref[0, :] = (rowmax * (1.0 / 127.0))[:, 0]
        b16_ref[:, :] = b_ref[:, :].astype(jnp.bfloat16)

        pl.semaphore_wait(barrier_sem, 4)

        def copy(src_slot, dst_slot, sem_idx, target):
            chunk = pltpu.make_async_remote_copy(
                src_ref=comm_ref.at[src_slot],
                dst_ref=comm_ref.at[dst_slot],
                send_sem=ss.at[sem_idx],
                recv_sem=rs.at[sem_idx],
                device_id=(target,),
                device_id_type=pl.DeviceIdType.MESH,
            )
            scale = pltpu.make_async_remote_copy(
                src_ref=scale_ref.at[src_slot],
                dst_ref=scale_ref.at[dst_slot],
                send_sem=ss2.at[sem_idx],
                recv_sem=rs2.at[sem_idx],
                device_id=(target,),
                device_id_type=pl.DeviceIdType.MESH,
            )
            chunk.start()
            scale.start()
            return chunk, scale

        def block(slot, origin):
            deq = jnp.dot(
                comm_ref[slot].astype(jnp.bfloat16),
                b16_ref[:, :],
                preferred_element_type=jnp.float32,
            )
            out_ref[pl.ds(origin * m_per, m_per), :] = (
                scale_ref[slot, :][:, None] * deq
            )

        p1x = copy(0, 1, 0, nx)
        p1y = copy(0, 2, 1, ny)
        p1z = copy(0, 3, 2, nz)
        p1a = copy(0, 4, 3, anti)

        block(0, my)

        p1x[0].wait_recv()
        p1x[1].wait_recv()
        p2z = copy(1, 7, 6, nz)
        block(1, nx)

        p1y[0].wait_recv()
        p1y[1].wait_recv()
        p2x = copy(2, 5, 4, nx)
        block(2, ny)

        p1z[0].wait_recv()
        p1z[1].wait_recv()
        p2y = copy(3, 6, 5, ny)
        block(3, nz)

        p1a[0].wait_recv()
        p1a[1].wait_recv()
        block(4, anti)

        p2x[0].wait_recv()
        p2x[1].wait_recv()
        block(5, dxy)
        p2y[0].wait_recv()
        p2y[1].wait_recv()
        block(6, dyz)
        p2z[0].wait_recv()
        p2z[1].wait_recv()
        block(7, dzx)

        for pair in (p1x, p1y, p1z, p1a, p2x, p2y, p2z):
            pair[0].wait_send()
            pair[1].wait_send()

    return pl.pallas_call(
        body,
        out_shape=jax.ShapeDtypeStruct((N_DEV * m_per, n), jnp.float32),
        in_specs=[
            pl.BlockSpec(memory_space=pltpu.VMEM),
            pl.BlockSpec(memory_space=pltpu.VMEM),
        ],
        out_specs=pl.BlockSpec(memory_space=pltpu.VMEM),
        scratch_shapes=[
            pltpu.VMEM((N_DEV, m_per, k), jnp.int8),
            pltpu.VMEM((N_DEV, m_per), jnp.float32),
            pltpu.VMEM((k, n), jnp.bfloat16),
            pltpu.SemaphoreType.DMA((7,)),
            pltpu.SemaphoreType.DMA((7,)),
            pltpu.SemaphoreType.DMA((7,)),
            pltpu.SemaphoreType.DMA((7,)),
        ],
        compiler_params=pltpu.CompilerParams(collective_id=0),
    )(A, B)
